# baseline (speedup 1.0000x reference)
import jax
import jax.numpy as jnp
from jax.experimental import pallas as pl
from jax.experimental.pallas import tpu as pltpu


def kernel(log_conditionals, u):
    B, V = log_conditionals.shape
    VB = 20000
    SH = 16
    nb = pl.cdiv(V, VB)
    tail = V - (nb - 1) * VB
    assert tail % SH == 0

    def body(l_ref, u_ref, o_ref, rmax, ridx):
        i = pl.program_id(0)

        @pl.when(i == 0)
        def _init():
            rmax[:] = jnp.full_like(rmax[:], -jnp.inf)
            ridx[:] = jnp.zeros_like(ridx[:])

        iota8 = jax.lax.broadcasted_iota(jnp.int32, (SH, B), 0)

        def scan_strips(n_strips):
            cm = rmax[:]
            ci = ridx[:]
            base = i * VB
            for k in range(n_strips):
                off = k * SH
                s = l_ref[off:off + SH, :] - jnp.log(-jnp.log(u_ref[off:off + SH, :]))
                idx = iota8 + (base + off)
                upd = s > cm
                cm = jnp.where(upd, s, cm)
                ci = jnp.where(upd, idx, ci)
            rmax[:] = cm
            ridx[:] = ci

        @pl.when(i < nb - 1)
        def _full():
            scan_strips(VB // SH)

        @pl.when(i == nb - 1)
        def _last():
            scan_strips(tail // SH)
            w = SH
            while w > 1:
                h = w // 2
                av, bv = rmax[0:h, :], rmax[h:w, :]
                ai, bi = ridx[0:h, :], ridx[h:w, :]
                take_b = (bv > av) | ((bv == av) & (bi < ai))
                rmax[0:h, :] = jnp.where(take_b, bv, av)
                ridx[0:h, :] = jnp.where(take_b, bi, ai)
                w = h
            o_ref[:] = ridx[0:1, :]

    out = pl.pallas_call(
        body,
        grid=(nb,),
        in_specs=[
            pl.BlockSpec((VB, B), lambda i: (i, 0)),
            pl.BlockSpec((VB, B), lambda i: (i, 0)),
        ],
        out_specs=pl.BlockSpec((1, B), lambda i: (0, 0)),
        out_shape=jax.ShapeDtypeStruct((1, B), jnp.int32),
        scratch_shapes=[
            pltpu.VMEM((SH, B), jnp.float32),
            pltpu.VMEM((SH, B), jnp.int32),
        ],
        compiler_params=pltpu.CompilerParams(
            dimension_semantics=("arbitrary",),
        ),
    )(log_conditionals.T, u.T)
    return out.reshape(B)


# speedup vs baseline: 1.0559x; 1.0559x over previous
import jax
import jax.numpy as jnp
from jax.experimental import pallas as pl
from jax.experimental.pallas import tpu as pltpu


def kernel(log_conditionals, u):
    B, V = log_conditionals.shape
    VB = 12800
    SH = 16
    nb = pl.cdiv(V, VB)
    tail = V - (nb - 1) * VB
    assert tail % SH == 0

    def body(l_ref, u_ref, o_ref, rmax, ridx):
        i = pl.program_id(0)

        @pl.when(i == 0)
        def _init():
            rmax[:] = jnp.full_like(rmax[:], -jnp.inf)
            ridx[:] = jnp.zeros_like(ridx[:])

        iota8 = jax.lax.broadcasted_iota(jnp.int32, (SH, B), 0)

        def scan_strips(n_strips):
            cm = rmax[:]
            ci = ridx[:]
            base = i * VB
            for k in range(n_strips):
                off = k * SH
                s = l_ref[off:off + SH, :] - jnp.log(-jnp.log(u_ref[off:off + SH, :]))
                idx = iota8 + (base + off)
                upd = s > cm
                cm = jnp.where(upd, s, cm)
                ci = jnp.where(upd, idx, ci)
            rmax[:] = cm
            ridx[:] = ci

        @pl.when(i < nb - 1)
        def _full():
            scan_strips(VB // SH)

        @pl.when(i == nb - 1)
        def _last():
            scan_strips(tail // SH)
            w = SH
            while w > 1:
                h = w // 2
                av, bv = rmax[0:h, :], rmax[h:w, :]
                ai, bi = ridx[0:h, :], ridx[h:w, :]
                take_b = (bv > av) | ((bv == av) & (bi < ai))
                rmax[0:h, :] = jnp.where(take_b, bv, av)
                ridx[0:h, :] = jnp.where(take_b, bi, ai)
                w = h
            o_ref[:] = ridx[0:1, :]

    out = pl.pallas_call(
        body,
        grid=(nb,),
        in_specs=[
            pl.BlockSpec((VB, B), lambda i: (i, 0)),
            pl.BlockSpec((VB, B), lambda i: (i, 0)),
        ],
        out_specs=pl.BlockSpec((1, B), lambda i: (0, 0)),
        out_shape=jax.ShapeDtypeStruct((1, B), jnp.int32),
        scratch_shapes=[
            pltpu.VMEM((SH, B), jnp.float32),
            pltpu.VMEM((SH, B), jnp.int32),
        ],
        compiler_params=pltpu.CompilerParams(
            dimension_semantics=("arbitrary",),
        ),
    )(log_conditionals.T, u.T)
    return out.reshape(B)
